# baseline (speedup 1.0000x reference)
import functools

import jax
import jax.numpy as jnp
from jax import lax
from jax.experimental import pallas as pl
from jax.experimental.pallas import tpu as pltpu
from jax.experimental.pallas import tpu_sc as plsc

NUM_USERS = 1000000
NUM_ITEMS = 1000000
EMB_DIM = 32
BATCH = 16384

NC = 2
NS = 16
NW = NC * NS
BPW = BATCH // NW
LANES = 16
GROUPS = BPW // LANES

_mesh = plsc.VectorSubcoreMesh(
    core_axis_name="c", subcore_axis_name="s", num_cores=NC, num_subcores=NS
)


@functools.partial(
    pl.kernel,
    out_type=jax.ShapeDtypeStruct((BATCH,), jnp.float32),
    mesh=_mesh,
    scratch_types=[
        pltpu.VMEM((BPW,), jnp.int32),
        pltpu.VMEM((BPW,), jnp.int32),
        pltpu.VMEM((EMB_DIM, BPW), jnp.float32),
        pltpu.VMEM((EMB_DIM, BPW), jnp.float32),
        pltpu.VMEM((BPW,), jnp.float32),
        pltpu.SemaphoreType.DMA,
    ],
    compiler_params=pltpu.CompilerParams(
        needs_layout_passes=False, use_tc_tiling_on_sc=False),
)
def _lightgcn_sc(uid_hbm, iid_hbm, utab_hbm, itab_hbm, out_hbm,
                 uidx_v, iidx_v, u_gath, i_gath, out_v, sem):
    wid = lax.axis_index("s") * NC + lax.axis_index("c")
    base = wid * BPW

    pltpu.sync_copy(uid_hbm.at[pl.ds(base, BPW)], uidx_v)
    pltpu.sync_copy(iid_hbm.at[pl.ds(base, BPW)], iidx_v)

    copies = []
    for d in range(EMB_DIM):
        copies.append(
            pltpu.async_copy(utab_hbm.at[d].at[uidx_v], u_gath.at[d], sem))
        copies.append(
            pltpu.async_copy(itab_hbm.at[d].at[iidx_v], i_gath.at[d], sem))
    for cp in copies:
        cp.wait()

    def group_body(g, carry):
        sl = pl.ds(g * LANES, LANES)
        acc = jnp.zeros((LANES,), jnp.float32)
        for d in range(EMB_DIM):
            acc = acc + u_gath[d, sl] * i_gath[d, sl]
        out_v[sl] = acc
        return carry

    lax.fori_loop(0, GROUPS, group_body, None)

    pltpu.sync_copy(out_v, out_hbm.at[pl.ds(base, BPW)])


def kernel(user_ids, item_ids, user_embeddings, item_embeddings):
    return _lightgcn_sc(
        user_ids.astype(jnp.int32),
        item_ids.astype(jnp.int32),
        user_embeddings.T,
        item_embeddings.T,
    )


# speedup vs baseline: 2.1918x; 2.1918x over previous
import functools

import jax
import jax.numpy as jnp
from jax import lax
from jax.experimental import pallas as pl
from jax.experimental.pallas import tpu as pltpu
from jax.experimental.pallas import tpu_sc as plsc

NUM_ROWS = 1000000
EMB_DIM = 32
BATCH = 16384

LINE = 128
ROWS_PER_BLOCK = 512
NBLOCKS = -(-NUM_ROWS // ROWS_PER_BLOCK)
NLINES = NBLOCKS * LINE

NC = 2
NS = 16
NW = NC * NS
BPW = BATCH // NW
CHUNK = 128
LANES = 16
PASS = 256
NPASS = BPW // PASS
PASS_CHUNKS = PASS // CHUNK
PASS_GROUPS = PASS // LANES


def _repack_lines(tab_t):
    def body(x_ref, o_ref):
        x = x_ref[...]
        for k in range(4):
            o_ref[:, 32 * k:32 * (k + 1)] = x[:, 128 * k:128 * (k + 1)].T

    return pl.pallas_call(
        body,
        grid=(NBLOCKS,),
        in_specs=[pl.BlockSpec((EMB_DIM, ROWS_PER_BLOCK), lambda i: (0, i))],
        out_specs=pl.BlockSpec((LINE, LINE), lambda i: (i, 0)),
        out_shape=jax.ShapeDtypeStruct((NLINES, LINE), jnp.float32),
    )(tab_t)


_mesh = plsc.VectorSubcoreMesh(
    core_axis_name="c", subcore_axis_name="s", num_cores=NC, num_subcores=NS
)


@functools.partial(
    pl.kernel,
    out_type=jax.ShapeDtypeStruct((BATCH,), jnp.float32),
    mesh=_mesh,
    scratch_types=[
        pltpu.VMEM((BPW,), jnp.int32),
        pltpu.VMEM((BPW,), jnp.int32),
        pltpu.VMEM((BPW,), jnp.int32),
        pltpu.VMEM((BPW,), jnp.int32),
        pltpu.VMEM((PASS, LINE), jnp.float32),
        pltpu.VMEM((PASS, LINE), jnp.float32),
        pltpu.VMEM((BPW,), jnp.float32),
        pltpu.SemaphoreType.DMA,
    ],
    compiler_params=pltpu.CompilerParams(
        needs_layout_passes=False, use_tc_tiling_on_sc=True),
)
def _lightgcn_sc(uid_hbm, iid_hbm, utab_hbm, itab_hbm, out_hbm,
                 uidx_v, iidx_v, uline_v, iline_v, urows_v, irows_v,
                 out_v, sem):
    wid = lax.axis_index("s") * NC + lax.axis_index("c")
    base = wid * BPW

    pltpu.sync_copy(uid_hbm.at[pl.ds(base, BPW)], uidx_v)
    pltpu.sync_copy(iid_hbm.at[pl.ds(base, BPW)], iidx_v)

    def line_body(k, carry):
        sl = pl.ds(k * LANES, LANES)
        uid = uidx_v[sl]
        iid = iidx_v[sl]
        uline_v[sl] = lax.shift_left(lax.shift_right_logical(uid, 9), 7) + \
            jnp.bitwise_and(uid, 127)
        iline_v[sl] = lax.shift_left(lax.shift_right_logical(iid, 9), 7) + \
            jnp.bitwise_and(iid, 127)
        return carry

    lax.fori_loop(0, BPW // LANES, line_body, None)

    lane_iota = lax.iota(jnp.int32, LANES)

    for p in range(NPASS):
        copies = []
        for j in range(PASS_CHUNKS):
            src = pl.ds(p * PASS + j * CHUNK, CHUNK)
            dst = pl.ds(j * CHUNK, CHUNK)
            copies.append(
                pltpu.async_copy(utab_hbm.at[uline_v.at[src]],
                                 urows_v.at[dst], sem))
            copies.append(
                pltpu.async_copy(itab_hbm.at[iline_v.at[src]],
                                 irows_v.at[dst], sem))
        for cp in copies:
            cp.wait()

        def group_body(g, carry):
            row0 = g * LANES
            row_idx = row0 + lane_iota
            gsl = pl.ds(p * PASS + row0, LANES)
            ucol0 = lax.shift_left(
                jnp.bitwise_and(lax.shift_right_logical(uidx_v[gsl], 7), 3), 5)
            icol0 = lax.shift_left(
                jnp.bitwise_and(lax.shift_right_logical(iidx_v[gsl], 7), 3), 5)
            acc = jnp.zeros((LANES,), jnp.float32)
            for d in range(EMB_DIM):
                u = plsc.load_gather(urows_v, [row_idx, ucol0 + d])
                v = plsc.load_gather(irows_v, [row_idx, icol0 + d])
                acc = acc + u * v
            out_v[gsl] = acc
            return carry

        lax.fori_loop(0, PASS_GROUPS, group_body, None)

    pltpu.sync_copy(out_v, out_hbm.at[pl.ds(base, BPW)])


def kernel(user_ids, item_ids, user_embeddings, item_embeddings):
    u_lines = _repack_lines(user_embeddings.T)
    i_lines = _repack_lines(item_embeddings.T)
    return _lightgcn_sc(
        user_ids.astype(jnp.int32),
        item_ids.astype(jnp.int32),
        u_lines,
        i_lines,
    )


# speedup vs baseline: 9.5677x; 4.3652x over previous
import functools

import jax
import jax.numpy as jnp
from jax import lax
from jax.experimental import pallas as pl
from jax.experimental.pallas import tpu as pltpu
from jax.experimental.pallas import tpu_sc as plsc

NUM_ROWS = 1000000
EMB_DIM = 32
BATCH = 16384

LINE = 128
ROWS_PER_BLOCK = 8192
NBLOCKS = -(-NUM_ROWS // ROWS_PER_BLOCK)
NLINES = NBLOCKS * ROWS_PER_BLOCK // 4

NC = 2
NS = 16
NW = NC * NS
BPW = BATCH // NW
CHUNK = 128
LANES = 16
PASS = 256
NPASS = BPW // PASS
PASS_CHUNKS = PASS // CHUNK
PASS_GROUPS = PASS // LANES


def _repack_lines(tab_t):
    def body(x_ref, eye_ref, o_ref):
        x = x_ref[...]
        eye = eye_ref[...]
        y = lax.dot_general(x, eye, (((0,), (0,)), ((), ())),
                            preferred_element_type=jnp.float32)
        for b in range(ROWS_PER_BLOCK // 512):
            for k in range(4):
                o_ref[128 * b:128 * (b + 1), 32 * k:32 * (k + 1)] = (
                    y[512 * b + 128 * k:512 * b + 128 * (k + 1), :])

    return pl.pallas_call(
        body,
        grid=(NBLOCKS,),
        in_specs=[
            pl.BlockSpec((EMB_DIM, ROWS_PER_BLOCK), lambda i: (0, i)),
            pl.BlockSpec((EMB_DIM, EMB_DIM), lambda i: (0, 0)),
        ],
        out_specs=pl.BlockSpec((ROWS_PER_BLOCK // 4, LINE), lambda i: (i, 0)),
        out_shape=jax.ShapeDtypeStruct((NLINES, LINE), jnp.float32),
    )(tab_t, jnp.eye(EMB_DIM, dtype=jnp.float32))


_mesh = plsc.VectorSubcoreMesh(
    core_axis_name="c", subcore_axis_name="s", num_cores=NC, num_subcores=NS
)


@functools.partial(
    pl.kernel,
    out_type=jax.ShapeDtypeStruct((BATCH,), jnp.float32),
    mesh=_mesh,
    scratch_types=[
        pltpu.VMEM((BPW,), jnp.int32),
        pltpu.VMEM((BPW,), jnp.int32),
        pltpu.VMEM((BPW,), jnp.int32),
        pltpu.VMEM((BPW,), jnp.int32),
        pltpu.VMEM((PASS, LINE), jnp.float32),
        pltpu.VMEM((PASS, LINE), jnp.float32),
        pltpu.VMEM((BPW,), jnp.float32),
        pltpu.SemaphoreType.DMA,
    ],
    compiler_params=pltpu.CompilerParams(
        needs_layout_passes=False, use_tc_tiling_on_sc=True),
)
def _lightgcn_sc(uid_hbm, iid_hbm, utab_hbm, itab_hbm, out_hbm,
                 uidx_v, iidx_v, uline_v, iline_v, urows_v, irows_v,
                 out_v, sem):
    wid = lax.axis_index("s") * NC + lax.axis_index("c")
    base = wid * BPW

    pltpu.sync_copy(uid_hbm.at[pl.ds(base, BPW)], uidx_v)
    pltpu.sync_copy(iid_hbm.at[pl.ds(base, BPW)], iidx_v)

    def line_body(k, carry):
        sl = pl.ds(k * LANES, LANES)
        uid = uidx_v[sl]
        iid = iidx_v[sl]
        uline_v[sl] = lax.shift_left(lax.shift_right_logical(uid, 9), 7) + \
            jnp.bitwise_and(uid, 127)
        iline_v[sl] = lax.shift_left(lax.shift_right_logical(iid, 9), 7) + \
            jnp.bitwise_and(iid, 127)
        return carry

    lax.fori_loop(0, BPW // LANES, line_body, None)

    lane_iota = lax.iota(jnp.int32, LANES)

    for p in range(NPASS):
        copies = []
        for j in range(PASS_CHUNKS):
            src = pl.ds(p * PASS + j * CHUNK, CHUNK)
            dst = pl.ds(j * CHUNK, CHUNK)
            copies.append(
                pltpu.async_copy(utab_hbm.at[uline_v.at[src]],
                                 urows_v.at[dst], sem))
            copies.append(
                pltpu.async_copy(itab_hbm.at[iline_v.at[src]],
                                 irows_v.at[dst], sem))
        for cp in copies:
            cp.wait()

        def group_body(g, carry):
            row0 = g * LANES
            row_idx = row0 + lane_iota
            gsl = pl.ds(p * PASS + row0, LANES)
            ucol0 = lax.shift_left(
                jnp.bitwise_and(lax.shift_right_logical(uidx_v[gsl], 7), 3), 5)
            icol0 = lax.shift_left(
                jnp.bitwise_and(lax.shift_right_logical(iidx_v[gsl], 7), 3), 5)
            acc = jnp.zeros((LANES,), jnp.float32)
            for d in range(EMB_DIM):
                u = plsc.load_gather(urows_v, [row_idx, ucol0 + d])
                v = plsc.load_gather(irows_v, [row_idx, icol0 + d])
                acc = acc + u * v
            out_v[gsl] = acc
            return carry

        lax.fori_loop(0, PASS_GROUPS, group_body, None)

    pltpu.sync_copy(out_v, out_hbm.at[pl.ds(base, BPW)])


def kernel(user_ids, item_ids, user_embeddings, item_embeddings):
    u_lines = _repack_lines(user_embeddings.T)
    i_lines = _repack_lines(item_embeddings.T)
    return _lightgcn_sc(
        user_ids.astype(jnp.int32),
        item_ids.astype(jnp.int32),
        u_lines,
        i_lines,
    )


# speedup vs baseline: 16.1972x; 1.6929x over previous
import functools

import jax
import jax.numpy as jnp
from jax import lax
from jax.experimental import pallas as pl
from jax.experimental.pallas import tpu as pltpu
from jax.experimental.pallas import tpu_sc as plsc

NUM_ROWS = 1000000
EMB_DIM = 32
BATCH = 16384

LINE = 128
ROWS_PER_BLOCK = 8192
NBLOCKS = -(-NUM_ROWS // ROWS_PER_BLOCK)
NLINES = NBLOCKS * ROWS_PER_BLOCK // 4

NC = 2
NS = 16
NW = NC * NS
BPW = BATCH // NW
CHUNK = 128
LANES = 16
PASS = 256
NPASS = BPW // PASS
PASS_CHUNKS = PASS // CHUNK
PASS_GROUPS = PASS // LANES


def _repack_lines(tab_t):
    def body(x_ref, eye_ref, o_ref):
        del eye_ref
        x = x_ref[...]
        for b in range(ROWS_PER_BLOCK // 512):
            w = jnp.concatenate(
                [x[:, 512 * b + 128 * k:512 * b + 128 * (k + 1)]
                 for k in range(4)], axis=0)
            o_ref[128 * b:128 * (b + 1), :] = w.T

    return pl.pallas_call(
        body,
        grid=(NBLOCKS,),
        in_specs=[
            pl.BlockSpec((EMB_DIM, ROWS_PER_BLOCK), lambda i: (0, i)),
            pl.BlockSpec((EMB_DIM, EMB_DIM), lambda i: (0, 0)),
        ],
        out_specs=pl.BlockSpec((ROWS_PER_BLOCK // 4, LINE), lambda i: (i, 0)),
        out_shape=jax.ShapeDtypeStruct((NLINES, LINE), jnp.float32),
    )(tab_t, jnp.eye(EMB_DIM, dtype=jnp.float32))


_mesh = plsc.VectorSubcoreMesh(
    core_axis_name="c", subcore_axis_name="s", num_cores=NC, num_subcores=NS
)


@functools.partial(
    pl.kernel,
    out_type=jax.ShapeDtypeStruct((BATCH,), jnp.float32),
    mesh=_mesh,
    scratch_types=[
        pltpu.VMEM((BPW,), jnp.int32),
        pltpu.VMEM((BPW,), jnp.int32),
        pltpu.VMEM((BPW,), jnp.int32),
        pltpu.VMEM((BPW,), jnp.int32),
        pltpu.VMEM((PASS, LINE), jnp.float32),
        pltpu.VMEM((PASS, LINE), jnp.float32),
        pltpu.VMEM((BPW,), jnp.float32),
        pltpu.SemaphoreType.DMA,
    ],
    compiler_params=pltpu.CompilerParams(
        needs_layout_passes=False, use_tc_tiling_on_sc=True),
)
def _lightgcn_sc(uid_hbm, iid_hbm, utab_hbm, itab_hbm, out_hbm,
                 uidx_v, iidx_v, uline_v, iline_v, urows_v, irows_v,
                 out_v, sem):
    wid = lax.axis_index("s") * NC + lax.axis_index("c")
    base = wid * BPW

    pltpu.sync_copy(uid_hbm.at[pl.ds(base, BPW)], uidx_v)
    pltpu.sync_copy(iid_hbm.at[pl.ds(base, BPW)], iidx_v)

    def line_body(k, carry):
        sl = pl.ds(k * LANES, LANES)
        uid = uidx_v[sl]
        iid = iidx_v[sl]
        uline_v[sl] = lax.shift_left(lax.shift_right_logical(uid, 9), 7) + \
            jnp.bitwise_and(uid, 127)
        iline_v[sl] = lax.shift_left(lax.shift_right_logical(iid, 9), 7) + \
            jnp.bitwise_and(iid, 127)
        return carry

    lax.fori_loop(0, BPW // LANES, line_body, None)

    lane_iota = lax.iota(jnp.int32, LANES)

    for p in range(NPASS):
        copies = []
        for j in range(PASS_CHUNKS):
            src = pl.ds(p * PASS + j * CHUNK, CHUNK)
            dst = pl.ds(j * CHUNK, CHUNK)
            copies.append(
                pltpu.async_copy(utab_hbm.at[uline_v.at[src]],
                                 urows_v.at[dst], sem))
            copies.append(
                pltpu.async_copy(itab_hbm.at[iline_v.at[src]],
                                 irows_v.at[dst], sem))
        for cp in copies:
            cp.wait()

        def group_body(g, carry):
            row0 = g * LANES
            row_idx = row0 + lane_iota
            gsl = pl.ds(p * PASS + row0, LANES)
            ucol0 = lax.shift_left(
                jnp.bitwise_and(lax.shift_right_logical(uidx_v[gsl], 7), 3), 5)
            icol0 = lax.shift_left(
                jnp.bitwise_and(lax.shift_right_logical(iidx_v[gsl], 7), 3), 5)
            acc = jnp.zeros((LANES,), jnp.float32)
            for d in range(EMB_DIM):
                u = plsc.load_gather(urows_v, [row_idx, ucol0 + d])
                v = plsc.load_gather(irows_v, [row_idx, icol0 + d])
                acc = acc + u * v
            out_v[gsl] = acc
            return carry

        lax.fori_loop(0, PASS_GROUPS, group_body, None)

    pltpu.sync_copy(out_v, out_hbm.at[pl.ds(base, BPW)])


def kernel(user_ids, item_ids, user_embeddings, item_embeddings):
    u_lines = _repack_lines(user_embeddings.T)
    i_lines = _repack_lines(item_embeddings.T)
    return _lightgcn_sc(
        user_ids.astype(jnp.int32),
        item_ids.astype(jnp.int32),
        u_lines,
        i_lines,
    )


# speedup vs baseline: 21.4055x; 1.3216x over previous
import functools

import jax
import jax.numpy as jnp
from jax import lax
from jax.experimental import pallas as pl
from jax.experimental.pallas import tpu as pltpu
from jax.experimental.pallas import tpu_sc as plsc

NUM_ROWS = 1000000
EMB_DIM = 32
BATCH = 16384

LINE = 128
ROWS_PER_BLOCK = 16384
NBLOCKS = -(-NUM_ROWS // ROWS_PER_BLOCK)
NLINES = NBLOCKS * ROWS_PER_BLOCK // 4

NC = 2
NS = 16
NW = NC * NS
BPW = BATCH // NW
CHUNK = 128
LANES = 16
PASS = 256
NPASS = BPW // PASS
PASS_CHUNKS = PASS // CHUNK
PASS_GROUPS = PASS // LANES


def _repack_lines(tab_t):
    def body(x_ref, eye_ref, o_ref):
        del eye_ref
        x = x_ref[...]
        for b in range(ROWS_PER_BLOCK // 512):
            w = jnp.concatenate(
                [x[:, 512 * b + 128 * k:512 * b + 128 * (k + 1)]
                 for k in range(4)], axis=0)
            o_ref[128 * b:128 * (b + 1), :] = w.T

    return pl.pallas_call(
        body,
        grid=(NBLOCKS,),
        in_specs=[
            pl.BlockSpec((EMB_DIM, ROWS_PER_BLOCK), lambda i: (0, i)),
            pl.BlockSpec((EMB_DIM, EMB_DIM), lambda i: (0, 0)),
        ],
        out_specs=pl.BlockSpec((ROWS_PER_BLOCK // 4, LINE), lambda i: (i, 0)),
        out_shape=jax.ShapeDtypeStruct((NLINES, LINE), jnp.float32),
    )(tab_t, jnp.eye(EMB_DIM, dtype=jnp.float32))


_mesh = plsc.VectorSubcoreMesh(
    core_axis_name="c", subcore_axis_name="s", num_cores=NC, num_subcores=NS
)


@functools.partial(
    pl.kernel,
    out_type=jax.ShapeDtypeStruct((BATCH,), jnp.float32),
    mesh=_mesh,
    scratch_types=[
        pltpu.VMEM((BPW,), jnp.int32),
        pltpu.VMEM((BPW,), jnp.int32),
        pltpu.VMEM((BPW,), jnp.int32),
        pltpu.VMEM((BPW,), jnp.int32),
        pltpu.VMEM((PASS, LINE), jnp.float32),
        pltpu.VMEM((PASS, LINE), jnp.float32),
        pltpu.VMEM((BPW,), jnp.float32),
        pltpu.SemaphoreType.DMA,
    ],
    compiler_params=pltpu.CompilerParams(
        needs_layout_passes=False, use_tc_tiling_on_sc=True),
)
def _lightgcn_sc(uid_hbm, iid_hbm, utab_hbm, itab_hbm, out_hbm,
                 uidx_v, iidx_v, uline_v, iline_v, urows_v, irows_v,
                 out_v, sem):
    wid = lax.axis_index("s") * NC + lax.axis_index("c")
    base = wid * BPW

    pltpu.sync_copy(uid_hbm.at[pl.ds(base, BPW)], uidx_v)
    pltpu.sync_copy(iid_hbm.at[pl.ds(base, BPW)], iidx_v)

    def line_body(k, carry):
        sl = pl.ds(k * LANES, LANES)
        uid = uidx_v[sl]
        iid = iidx_v[sl]
        uline_v[sl] = lax.shift_left(lax.shift_right_logical(uid, 9), 7) + \
            jnp.bitwise_and(uid, 127)
        iline_v[sl] = lax.shift_left(lax.shift_right_logical(iid, 9), 7) + \
            jnp.bitwise_and(iid, 127)
        return carry

    lax.fori_loop(0, BPW // LANES, line_body, None)

    lane_iota = lax.iota(jnp.int32, LANES)

    for p in range(NPASS):
        copies = []
        for j in range(PASS_CHUNKS):
            src = pl.ds(p * PASS + j * CHUNK, CHUNK)
            dst = pl.ds(j * CHUNK, CHUNK)
            copies.append(
                pltpu.async_copy(utab_hbm.at[uline_v.at[src]],
                                 urows_v.at[dst], sem))
            copies.append(
                pltpu.async_copy(itab_hbm.at[iline_v.at[src]],
                                 irows_v.at[dst], sem))
        for cp in copies:
            cp.wait()

        def group_body(g, carry):
            row0 = g * LANES
            row_idx = row0 + lane_iota
            gsl = pl.ds(p * PASS + row0, LANES)
            ucol0 = lax.shift_left(
                jnp.bitwise_and(lax.shift_right_logical(uidx_v[gsl], 7), 3), 5)
            icol0 = lax.shift_left(
                jnp.bitwise_and(lax.shift_right_logical(iidx_v[gsl], 7), 3), 5)
            acc = jnp.zeros((LANES,), jnp.float32)
            for d in range(EMB_DIM):
                u = plsc.load_gather(urows_v, [row_idx, ucol0 + d])
                v = plsc.load_gather(irows_v, [row_idx, icol0 + d])
                acc = acc + u * v
            out_v[gsl] = acc
            return carry

        lax.fori_loop(0, PASS_GROUPS, group_body, None)

    pltpu.sync_copy(out_v, out_hbm.at[pl.ds(base, BPW)])


def kernel(user_ids, item_ids, user_embeddings, item_embeddings):
    u_lines = _repack_lines(user_embeddings.T)
    i_lines = _repack_lines(item_embeddings.T)
    return _lightgcn_sc(
        user_ids.astype(jnp.int32),
        item_ids.astype(jnp.int32),
        u_lines,
        i_lines,
    )


# speedup vs baseline: 24.3436x; 1.1373x over previous
import functools

import jax
import jax.numpy as jnp
from jax import lax
from jax.experimental import pallas as pl
from jax.experimental.pallas import tpu as pltpu
from jax.experimental.pallas import tpu_sc as plsc

NUM_ROWS = 1000000
EMB_DIM = 32
BATCH = 16384

LINE = 128
ROWS_PER_BLOCK = 32768
NBLOCKS = -(-NUM_ROWS // ROWS_PER_BLOCK)
NLINES = NBLOCKS * ROWS_PER_BLOCK // 4

NC = 2
NS = 16
NW = NC * NS
BPW = BATCH // NW
CHUNK = 128
LANES = 16
PASS = 256
NPASS = BPW // PASS
PASS_CHUNKS = PASS // CHUNK
PASS_GROUPS = PASS // LANES


def _repack_lines(tab_t):
    def body(x_ref, eye_ref, o_ref):
        del eye_ref
        x = x_ref[...]
        for b in range(ROWS_PER_BLOCK // 512):
            w = jnp.concatenate(
                [x[:, 512 * b + 128 * k:512 * b + 128 * (k + 1)]
                 for k in range(4)], axis=0)
            o_ref[128 * b:128 * (b + 1), :] = w.T

    return pl.pallas_call(
        body,
        grid=(NBLOCKS,),
        in_specs=[
            pl.BlockSpec((EMB_DIM, ROWS_PER_BLOCK), lambda i: (0, i)),
            pl.BlockSpec((EMB_DIM, EMB_DIM), lambda i: (0, 0)),
        ],
        out_specs=pl.BlockSpec((ROWS_PER_BLOCK // 4, LINE), lambda i: (i, 0)),
        out_shape=jax.ShapeDtypeStruct((NLINES, LINE), jnp.float32),
    )(tab_t, jnp.eye(EMB_DIM, dtype=jnp.float32))


_mesh = plsc.VectorSubcoreMesh(
    core_axis_name="c", subcore_axis_name="s", num_cores=NC, num_subcores=NS
)


@functools.partial(
    pl.kernel,
    out_type=jax.ShapeDtypeStruct((BATCH,), jnp.float32),
    mesh=_mesh,
    scratch_types=[
        pltpu.VMEM((BPW,), jnp.int32),
        pltpu.VMEM((BPW,), jnp.int32),
        pltpu.VMEM((BPW,), jnp.int32),
        pltpu.VMEM((BPW,), jnp.int32),
        pltpu.VMEM((PASS, LINE), jnp.float32),
        pltpu.VMEM((PASS, LINE), jnp.float32),
        pltpu.VMEM((BPW,), jnp.float32),
        pltpu.SemaphoreType.DMA,
    ],
    compiler_params=pltpu.CompilerParams(
        needs_layout_passes=False, use_tc_tiling_on_sc=True),
)
def _lightgcn_sc(uid_hbm, iid_hbm, utab_hbm, itab_hbm, out_hbm,
                 uidx_v, iidx_v, uline_v, iline_v, urows_v, irows_v,
                 out_v, sem):
    wid = lax.axis_index("s") * NC + lax.axis_index("c")
    base = wid * BPW

    pltpu.sync_copy(uid_hbm.at[pl.ds(base, BPW)], uidx_v)
    pltpu.sync_copy(iid_hbm.at[pl.ds(base, BPW)], iidx_v)

    def line_body(k, carry):
        sl = pl.ds(k * LANES, LANES)
        uid = uidx_v[sl]
        iid = iidx_v[sl]
        uline_v[sl] = lax.shift_left(lax.shift_right_logical(uid, 9), 7) + \
            jnp.bitwise_and(uid, 127)
        iline_v[sl] = lax.shift_left(lax.shift_right_logical(iid, 9), 7) + \
            jnp.bitwise_and(iid, 127)
        return carry

    lax.fori_loop(0, BPW // LANES, line_body, None)

    lane_iota = lax.iota(jnp.int32, LANES)

    for p in range(NPASS):
        copies = []
        for j in range(PASS_CHUNKS):
            src = pl.ds(p * PASS + j * CHUNK, CHUNK)
            dst = pl.ds(j * CHUNK, CHUNK)
            copies.append(
                pltpu.async_copy(utab_hbm.at[uline_v.at[src]],
                                 urows_v.at[dst], sem))
            copies.append(
                pltpu.async_copy(itab_hbm.at[iline_v.at[src]],
                                 irows_v.at[dst], sem))
        for cp in copies:
            cp.wait()

        def group_body(g, carry):
            row0 = g * LANES
            row_idx = row0 + lane_iota
            gsl = pl.ds(p * PASS + row0, LANES)
            ucol0 = lax.shift_left(
                jnp.bitwise_and(lax.shift_right_logical(uidx_v[gsl], 7), 3), 5)
            icol0 = lax.shift_left(
                jnp.bitwise_and(lax.shift_right_logical(iidx_v[gsl], 7), 3), 5)
            acc = jnp.zeros((LANES,), jnp.float32)
            for d in range(EMB_DIM):
                u = plsc.load_gather(urows_v, [row_idx, ucol0 + d])
                v = plsc.load_gather(irows_v, [row_idx, icol0 + d])
                acc = acc + u * v
            out_v[gsl] = acc
            return carry

        lax.fori_loop(0, PASS_GROUPS, group_body, None)

    pltpu.sync_copy(out_v, out_hbm.at[pl.ds(base, BPW)])


def kernel(user_ids, item_ids, user_embeddings, item_embeddings):
    u_lines = _repack_lines(user_embeddings.T)
    i_lines = _repack_lines(item_embeddings.T)
    return _lightgcn_sc(
        user_ids.astype(jnp.int32),
        item_ids.astype(jnp.int32),
        u_lines,
        i_lines,
    )


# speedup vs baseline: 24.5940x; 1.0103x over previous
import functools

import jax
import jax.numpy as jnp
from jax import lax
from jax.experimental import pallas as pl
from jax.experimental.pallas import tpu as pltpu
from jax.experimental.pallas import tpu_sc as plsc

NUM_ROWS = 1000000
EMB_DIM = 32
BATCH = 16384

LINE = 128
ROWS_PER_BLOCK = 65536
NBLOCKS = -(-NUM_ROWS // ROWS_PER_BLOCK)
NLINES = NBLOCKS * ROWS_PER_BLOCK // 4

NC = 2
NS = 16
NW = NC * NS
BPW = BATCH // NW
CHUNK = 128
LANES = 16
PASS = 256
NPASS = BPW // PASS
PASS_CHUNKS = PASS // CHUNK
PASS_GROUPS = PASS // LANES


def _repack_lines(tab_t):
    def body(x_ref, eye_ref, o_ref):
        del eye_ref
        x = x_ref[...]
        for b in range(ROWS_PER_BLOCK // 512):
            w = jnp.concatenate(
                [x[:, 512 * b + 128 * k:512 * b + 128 * (k + 1)]
                 for k in range(4)], axis=0)
            o_ref[128 * b:128 * (b + 1), :] = w.T

    return pl.pallas_call(
        body,
        grid=(NBLOCKS,),
        in_specs=[
            pl.BlockSpec((EMB_DIM, ROWS_PER_BLOCK), lambda i: (0, i)),
            pl.BlockSpec((EMB_DIM, EMB_DIM), lambda i: (0, 0)),
        ],
        out_specs=pl.BlockSpec((ROWS_PER_BLOCK // 4, LINE), lambda i: (i, 0)),
        out_shape=jax.ShapeDtypeStruct((NLINES, LINE), jnp.float32),
    )(tab_t, jnp.eye(EMB_DIM, dtype=jnp.float32))


_mesh = plsc.VectorSubcoreMesh(
    core_axis_name="c", subcore_axis_name="s", num_cores=NC, num_subcores=NS
)


@functools.partial(
    pl.kernel,
    out_type=jax.ShapeDtypeStruct((BATCH,), jnp.float32),
    mesh=_mesh,
    scratch_types=[
        pltpu.VMEM((BPW,), jnp.int32),
        pltpu.VMEM((BPW,), jnp.int32),
        pltpu.VMEM((BPW,), jnp.int32),
        pltpu.VMEM((BPW,), jnp.int32),
        pltpu.VMEM((PASS, LINE), jnp.float32),
        pltpu.VMEM((PASS, LINE), jnp.float32),
        pltpu.VMEM((BPW,), jnp.float32),
        pltpu.SemaphoreType.DMA,
    ],
    compiler_params=pltpu.CompilerParams(
        needs_layout_passes=False, use_tc_tiling_on_sc=True),
)
def _lightgcn_sc(uid_hbm, iid_hbm, utab_hbm, itab_hbm, out_hbm,
                 uidx_v, iidx_v, uline_v, iline_v, urows_v, irows_v,
                 out_v, sem):
    wid = lax.axis_index("s") * NC + lax.axis_index("c")
    base = wid * BPW

    pltpu.sync_copy(uid_hbm.at[pl.ds(base, BPW)], uidx_v)
    pltpu.sync_copy(iid_hbm.at[pl.ds(base, BPW)], iidx_v)

    def line_body(k, carry):
        sl = pl.ds(k * LANES, LANES)
        uid = uidx_v[sl]
        iid = iidx_v[sl]
        uline_v[sl] = lax.shift_left(lax.shift_right_logical(uid, 9), 7) + \
            jnp.bitwise_and(uid, 127)
        iline_v[sl] = lax.shift_left(lax.shift_right_logical(iid, 9), 7) + \
            jnp.bitwise_and(iid, 127)
        return carry

    lax.fori_loop(0, BPW // LANES, line_body, None)

    lane_iota = lax.iota(jnp.int32, LANES)

    for p in range(NPASS):
        copies = []
        for j in range(PASS_CHUNKS):
            src = pl.ds(p * PASS + j * CHUNK, CHUNK)
            dst = pl.ds(j * CHUNK, CHUNK)
            copies.append(
                pltpu.async_copy(utab_hbm.at[uline_v.at[src]],
                                 urows_v.at[dst], sem))
            copies.append(
                pltpu.async_copy(itab_hbm.at[iline_v.at[src]],
                                 irows_v.at[dst], sem))
        for cp in copies:
            cp.wait()

        def group_body(g, carry):
            row0 = g * LANES
            row_idx = row0 + lane_iota
            gsl = pl.ds(p * PASS + row0, LANES)
            ucol0 = lax.shift_left(
                jnp.bitwise_and(lax.shift_right_logical(uidx_v[gsl], 7), 3), 5)
            icol0 = lax.shift_left(
                jnp.bitwise_and(lax.shift_right_logical(iidx_v[gsl], 7), 3), 5)
            acc = jnp.zeros((LANES,), jnp.float32)
            for d in range(EMB_DIM):
                u = plsc.load_gather(urows_v, [row_idx, ucol0 + d])
                v = plsc.load_gather(irows_v, [row_idx, icol0 + d])
                acc = acc + u * v
            out_v[gsl] = acc
            return carry

        lax.fori_loop(0, PASS_GROUPS, group_body, None)

    pltpu.sync_copy(out_v, out_hbm.at[pl.ds(base, BPW)])


def kernel(user_ids, item_ids, user_embeddings, item_embeddings):
    u_lines = _repack_lines(user_embeddings.T)
    i_lines = _repack_lines(item_embeddings.T)
    return _lightgcn_sc(
        user_ids.astype(jnp.int32),
        item_ids.astype(jnp.int32),
        u_lines,
        i_lines,
    )
